# baseline (speedup 1.0000x reference)
import jax
import jax.numpy as jnp
from jax import lax
from jax.experimental import pallas as pl
from jax.experimental.pallas import tpu as pltpu
from jax.experimental.pallas import tpu_sc as plsc

NUM_USERS = 50000
NUM_ITEMS = 50000
NUM_GROUPS = 10000
NUM_UI = NUM_USERS + NUM_ITEMS
EMB = 128
UH_NNZ = 320000
FH_NNZ = 640000

_A_PER_TILE = UH_NNZ // 16
_A_CH = 80
_A_NCH = _A_PER_TILE // _A_CH


_GDN = lax.GatherDimensionNumbers(
    offset_dims=(), collapsed_slice_dims=(0,), start_index_map=(0,))


def _bcast_lane(vec, j):
    idx = jnp.full((16, 1), j, jnp.int32)
    return lax.gather(vec, idx, _GDN, (1,),
                      mode=lax.GatherScatterMode.PROMISE_IN_BOUNDS)


def _scale_chunk(rows_v, val_v, ch, width):
    for g in range(ch // 16):
        vals = val_v[pl.ds(g * 16, 16)]
        for j in range(16):
            vj = _bcast_lane(vals, j)
            jj = g * 16 + j
            for k in range(width // 16):
                sl = pl.ds(k * 16, 16)
                rows_v[jj, sl] = rows_v[jj, sl] * vj


def _group_spmm_body(table, cols, vals, rows, zeros, out,
                     acc, col_v, val_v, row_v, rows_v):
    c = lax.axis_index("c")
    s = lax.axis_index("s")
    @pl.when(s < 15)
    def _():
        pltpu.sync_copy(zeros.at[pl.ds(0, 624)], acc.at[pl.ds(s * 624, 624)])

    @pl.when(s == 15)
    def _():
        pltpu.sync_copy(zeros.at[pl.ds(0, 640)], acc.at[pl.ds(9360, 640)])

    plsc.subcore_barrier()

    base = c * UH_NNZ + s * _A_PER_TILE

    def chunk(i, _):
        lo = base + i * _A_CH
        pltpu.sync_copy(cols.at[pl.ds(lo, _A_CH)], col_v)
        pltpu.sync_copy(vals.at[pl.ds(lo, _A_CH)], val_v)
        pltpu.sync_copy(rows.at[pl.ds(lo, _A_CH)], row_v)
        pltpu.sync_copy(table.at[col_v], rows_v)
        _scale_chunk(rows_v, val_v, _A_CH, EMB)
        pltpu.sync_copy(rows_v, acc.at[row_v], add=True)
        return 0

    lax.fori_loop(0, _A_NCH, chunk, 0)
    plsc.subcore_barrier()

    @pl.when(s < 15)
    def _():
        pltpu.sync_copy(acc.at[pl.ds(s * 624, 624)],
                        out.at[c, pl.ds(s * 624, 624)])

    @pl.when(s == 15)
    def _():
        pltpu.sync_copy(acc.at[pl.ds(9360, 640)],
                        out.at[c, pl.ds(9360, 640)])


def _group_spmm(table, cols, vals, rows, zeros):
    mesh = plsc.VectorSubcoreMesh(core_axis_name="c", subcore_axis_name="s")
    return pl.kernel(
        _group_spmm_body,
        mesh=mesh,
        out_type=jax.ShapeDtypeStruct((2, NUM_GROUPS, EMB), jnp.float32),
        scratch_types=[
            pltpu.VMEM_SHARED((NUM_GROUPS, EMB), jnp.float32),
            pltpu.VMEM((_A_CH,), jnp.int32),
            pltpu.VMEM((_A_CH,), jnp.float32),
            pltpu.VMEM((_A_CH,), jnp.int32),
            pltpu.VMEM((_A_CH, EMB), jnp.float32),
        ],
    )(table, cols, vals, rows, zeros)


_F_CH = 80
_F_PER_TILE = FH_NNZ // 16
_F_NCH = _F_PER_TILE // _F_CH
_NB = 8
_BIN = 12504
_F_ZR0 = 784
_F_ZR15 = _BIN - 15 * _F_ZR0


def _fh_spmm_body(msg, cols, vals, rows, zeros, out,
                  acc, col_v, val_v, row_v, rows_v):
    c = lax.axis_index("c")
    s = lax.axis_index("s")

    def do_pass(p, _):
        b = c * (_NB // 2) + p

        @pl.when(s < 15)
        def _():
            pltpu.sync_copy(zeros.at[pl.ds(0, _F_ZR0)],
                            acc.at[pl.ds(s * _F_ZR0, _F_ZR0)])

        @pl.when(s == 15)
        def _():
            pltpu.sync_copy(zeros.at[pl.ds(0, _F_ZR15)],
                            acc.at[pl.ds(15 * _F_ZR0, _F_ZR15)])

        plsc.subcore_barrier()

        ebase = s * _F_PER_TILE
        rbase = b * _BIN

        def chunk(i, _):
            lo = ebase + i * _F_CH
            pltpu.sync_copy(cols.at[pl.ds(lo, _F_CH)], col_v)
            pltpu.sync_copy(vals.at[pl.ds(lo, _F_CH)], val_v)
            pltpu.sync_copy(rows.at[pl.ds(lo, _F_CH)], row_v)
            for g in range(_F_CH // 16):
                sl = pl.ds(g * 16, 16)
                rel = row_v[sl] - rbase
                inbin = (rel >= 0) & (rel < _BIN)
                val_v[sl] = jnp.where(inbin, val_v[sl], 0.0)
                row_v[sl] = jnp.minimum(jnp.maximum(rel, 0), _BIN - 1)
            pltpu.sync_copy(msg.at[col_v], rows_v)
            _scale_chunk(rows_v, val_v, _F_CH, EMB)
            pltpu.sync_copy(rows_v, acc.at[row_v], add=True)
            return 0

        lax.fori_loop(0, _F_NCH, chunk, 0)
        plsc.subcore_barrier()

        @pl.when(s < 15)
        def _():
            pltpu.sync_copy(acc.at[pl.ds(s * _F_ZR0, _F_ZR0)],
                            out.at[pl.ds(rbase + s * _F_ZR0, _F_ZR0)])

        @pl.when((s == 15) & (b < _NB - 1))
        def _():
            pltpu.sync_copy(acc.at[pl.ds(15 * _F_ZR0, _F_ZR15)],
                            out.at[pl.ds(rbase + 15 * _F_ZR0, _F_ZR15)])

        @pl.when((s == 15) & (b == _NB - 1))
        def _():
            last = NUM_UI - (_NB - 1) * _BIN - 15 * _F_ZR0
            pltpu.sync_copy(acc.at[pl.ds(15 * _F_ZR0, last)],
                            out.at[pl.ds(rbase + 15 * _F_ZR0, last)])

        return 0

    lax.fori_loop(0, _NB // 2, do_pass, 0)


def _fh_spmm(msg, cols, vals, rows, zeros):
    mesh = plsc.VectorSubcoreMesh(core_axis_name="c", subcore_axis_name="s")
    return pl.kernel(
        _fh_spmm_body,
        mesh=mesh,
        out_type=jax.ShapeDtypeStruct((NUM_UI, EMB), jnp.float32),
        scratch_types=[
            pltpu.VMEM_SHARED((_BIN, EMB), jnp.float32),
            pltpu.VMEM((_F_CH,), jnp.int32),
            pltpu.VMEM((_F_CH,), jnp.float32),
            pltpu.VMEM((_F_CH,), jnp.int32),
            pltpu.VMEM((_F_CH, EMB), jnp.float32),
        ],
    )(msg, cols, vals, rows, zeros)


def _agg_matmul_body(um_ref, im_ref, w_ref, b_ref, g_ref, msg_ref, gout_ref):
    x = jnp.concatenate([um_ref[...], im_ref[...]], axis=1)
    msg = (
        jax.lax.dot_general(
            x, w_ref[...], (((1,), (0,)), ((), ())),
            preferred_element_type=jnp.float32,
            precision=jax.lax.Precision.HIGHEST,
        )
        + b_ref[...]
    )
    msg_ref[...] = msg
    gout_ref[...] = g_ref[...] + msg


def _agg_matmul(user_msg, item_msg, w, b, g_acc):
    blk = 2000
    grid = (NUM_GROUPS // blk,)
    return pl.pallas_call(
        _agg_matmul_body,
        grid=grid,
        in_specs=[
            pl.BlockSpec((blk, EMB), lambda i: (i, 0)),
            pl.BlockSpec((blk, EMB), lambda i: (i, 0)),
            pl.BlockSpec((2 * EMB, EMB), lambda i: (0, 0)),
            pl.BlockSpec((1, EMB), lambda i: (0, 0)),
            pl.BlockSpec((blk, EMB), lambda i: (i, 0)),
        ],
        out_specs=[
            pl.BlockSpec((blk, EMB), lambda i: (i, 0)),
            pl.BlockSpec((blk, EMB), lambda i: (i, 0)),
        ],
        out_shape=[
            jax.ShapeDtypeStruct((NUM_GROUPS, EMB), jnp.float32),
            jax.ShapeDtypeStruct((NUM_GROUPS, EMB), jnp.float32),
        ],
    )(user_msg, item_msg, w, b, g_acc)


def _ui_sum_body(u_ref, i_ref, e1_ref, e2_ref, o_ref):
    half = pl.program_id(1)
    base = jnp.where(half == 0, u_ref[...], i_ref[...])
    o_ref[...] = base + e1_ref[...] + e2_ref[...]


def _ui_sum(user_emb, item_emb, emb1, emb2):
    blk = 2000
    nb = NUM_USERS // blk
    return pl.pallas_call(
        _ui_sum_body,
        grid=(nb, 2),
        in_specs=[
            pl.BlockSpec((blk, EMB), lambda i, h: (i, 0)),
            pl.BlockSpec((blk, EMB), lambda i, h: (i, 0)),
            pl.BlockSpec((blk, EMB), lambda i, h: (i + h * nb, 0)),
            pl.BlockSpec((blk, EMB), lambda i, h: (i + h * nb, 0)),
        ],
        out_specs=pl.BlockSpec((blk, EMB), lambda i, h: (i + h * nb, 0)),
        out_shape=jax.ShapeDtypeStruct((NUM_UI, EMB), jnp.float32),
    )(user_emb, item_emb, emb1, emb2)


def kernel(user_emb, item_emb, group_emb, W_agg, b_agg,
           uh_row, uh_col, uh_val, ih_row, ih_col, ih_val,
           fh_row, fh_col, fh_val):
    i32 = jnp.int32
    ui0 = jnp.concatenate([user_emb, item_emb], axis=0)
    a_cols = jnp.concatenate([uh_col.astype(i32),
                              ih_col.astype(i32) + NUM_USERS])
    a_vals = jnp.concatenate([uh_val, ih_val])
    a_rows = jnp.concatenate([uh_row.astype(i32), ih_row.astype(i32)])
    f_cols = fh_col.astype(i32)
    f_rows = fh_row.astype(i32)
    zeros_g = jnp.zeros((640, EMB), jnp.float32)
    zeros_f = jnp.zeros((_F_ZR0, EMB), jnp.float32)

    msgs = _group_spmm(ui0, a_cols, a_vals, a_rows, zeros_g)
    msg, g_acc = _agg_matmul(msgs[0], msgs[1], W_agg[0],
                             b_agg[0].reshape(1, EMB), group_emb)
    emb1 = _fh_spmm(msg, f_cols, fh_val, f_rows, zeros_f)

    msgs = _group_spmm(emb1, a_cols, a_vals, a_rows, zeros_g)
    msg, g_acc = _agg_matmul(msgs[0], msgs[1], W_agg[1],
                             b_agg[1].reshape(1, EMB), g_acc)
    emb2 = _fh_spmm(msg, f_cols, fh_val, f_rows, zeros_f)

    final_ui = _ui_sum(user_emb, item_emb, emb1, emb2)
    return (final_ui, g_acc)


# speedup vs baseline: 3.3963x; 3.3963x over previous
import jax
import jax.numpy as jnp
from jax import lax
from jax.experimental import pallas as pl
from jax.experimental.pallas import tpu as pltpu
from jax.experimental.pallas import tpu_sc as plsc

NUM_USERS = 50000
NUM_ITEMS = 50000
NUM_GROUPS = 10000
NUM_UI = NUM_USERS + NUM_ITEMS
EMB = 128
UH_NNZ = 320000
FH_NNZ = 640000

_A_PER_TILE = UH_NNZ // 16
_A_CH = 80
_A_NCH = _A_PER_TILE // _A_CH


_GDN = lax.GatherDimensionNumbers(
    offset_dims=(), collapsed_slice_dims=(0,), start_index_map=(0,))


def _bcast_lane(vec, j):
    idx = jnp.full((16, 1), j, jnp.int32)
    return lax.gather(vec, idx, _GDN, (1,),
                      mode=lax.GatherScatterMode.PROMISE_IN_BOUNDS)


def _scale_chunk(rows_v, val_v, ch, width):
    for g in range(ch // 16):
        vals = val_v[pl.ds(g * 16, 16)]
        for j in range(16):
            vj = _bcast_lane(vals, j)
            jj = g * 16 + j
            for k in range(width // 16):
                sl = pl.ds(k * 16, 16)
                rows_v[jj, sl] = rows_v[jj, sl] * vj


def _group_spmm_body(table, cols, vals, rows, zeros, out,
                     acc, col_v, val_v, row_v, rows_v):
    c = lax.axis_index("c")
    s = lax.axis_index("s")
    @pl.when(s < 15)
    def _():
        pltpu.sync_copy(zeros.at[pl.ds(0, 624)], acc.at[pl.ds(s * 624, 624)])

    @pl.when(s == 15)
    def _():
        pltpu.sync_copy(zeros.at[pl.ds(0, 640)], acc.at[pl.ds(9360, 640)])

    plsc.subcore_barrier()

    base = c * UH_NNZ + s * _A_PER_TILE

    def chunk(i, _):
        lo = base + i * _A_CH
        pltpu.sync_copy(cols.at[pl.ds(lo, _A_CH)], col_v)
        pltpu.sync_copy(vals.at[pl.ds(lo, _A_CH)], val_v)
        pltpu.sync_copy(rows.at[pl.ds(lo, _A_CH)], row_v)
        pltpu.sync_copy(table.at[col_v], rows_v)
        _scale_chunk(rows_v, val_v, _A_CH, EMB)
        pltpu.sync_copy(rows_v, acc.at[row_v], add=True)
        return 0

    lax.fori_loop(0, _A_NCH, chunk, 0)
    plsc.subcore_barrier()

    @pl.when(s < 15)
    def _():
        pltpu.sync_copy(acc.at[pl.ds(s * 624, 624)],
                        out.at[c, pl.ds(s * 624, 624)])

    @pl.when(s == 15)
    def _():
        pltpu.sync_copy(acc.at[pl.ds(9360, 640)],
                        out.at[c, pl.ds(9360, 640)])


def _group_spmm(table, cols, vals, rows, zeros):
    mesh = plsc.VectorSubcoreMesh(core_axis_name="c", subcore_axis_name="s")
    return pl.kernel(
        _group_spmm_body,
        mesh=mesh,
        out_type=jax.ShapeDtypeStruct((2, NUM_GROUPS, EMB), jnp.float32),
        scratch_types=[
            pltpu.VMEM_SHARED((NUM_GROUPS, EMB), jnp.float32),
            pltpu.VMEM((_A_CH,), jnp.int32),
            pltpu.VMEM((_A_CH,), jnp.float32),
            pltpu.VMEM((_A_CH,), jnp.int32),
            pltpu.VMEM((_A_CH, EMB), jnp.float32),
        ],
    )(table, cols, vals, rows, zeros)


_F_CH = 80
_F_PER_TILE = FH_NNZ // 16
_F_NCH = _F_PER_TILE // _F_CH
_NB = 8
_BIN = 12504
_F_ZR0 = 784
_F_ZR15 = _BIN - 15 * _F_ZR0


def _fh_spmm_body(msg, cols, vals, rows, bnds, zeros, out,
                  acc, bnd_v, col_v, val_v, row_v, rows_v):
    c = lax.axis_index("c")
    s = lax.axis_index("s")
    pltpu.sync_copy(bnds.at[pl.ds(0, 16)], bnd_v)

    def do_pass(p, _):
        b = c * (_NB // 2) + p

        @pl.when(s < 15)
        def _():
            pltpu.sync_copy(zeros.at[pl.ds(0, _F_ZR0)],
                            acc.at[pl.ds(s * _F_ZR0, _F_ZR0)])

        @pl.when(s == 15)
        def _():
            pltpu.sync_copy(zeros.at[pl.ds(0, _F_ZR15)],
                            acc.at[pl.ds(15 * _F_ZR0, _F_ZR15)])

        plsc.subcore_barrier()

        start = bnd_v[pl.ds(b, 1)][0]
        end = bnd_v[pl.ds(b + 1, 1)][0]
        astart = start // 8 * 8
        nch = (end - astart + _F_CH - 1) // _F_CH
        my_nch = (nch - s + 15) // 16
        rbase = b * _BIN

        def chunk(k, _):
            lo = pl.multiple_of(astart + (s + k * 16) * _F_CH, 8)
            pltpu.sync_copy(cols.at[pl.ds(lo, _F_CH)], col_v)
            pltpu.sync_copy(vals.at[pl.ds(lo, _F_CH)], val_v)
            pltpu.sync_copy(rows.at[pl.ds(lo, _F_CH)], row_v)
            for g in range(_F_CH // 16):
                sl = pl.ds(g * 16, 16)
                rel = row_v[sl] - rbase
                inbin = (rel >= 0) & (rel < _BIN)
                val_v[sl] = jnp.where(inbin, val_v[sl], 0.0)
                row_v[sl] = jnp.minimum(jnp.maximum(rel, 0), _BIN - 1)
            pltpu.sync_copy(msg.at[col_v], rows_v)
            _scale_chunk(rows_v, val_v, _F_CH, EMB)
            pltpu.sync_copy(rows_v, acc.at[row_v], add=True)
            return 0

        lax.fori_loop(0, my_nch, chunk, 0)
        plsc.subcore_barrier()

        @pl.when(s < 15)
        def _():
            pltpu.sync_copy(acc.at[pl.ds(s * _F_ZR0, _F_ZR0)],
                            out.at[pl.ds(rbase + s * _F_ZR0, _F_ZR0)])

        @pl.when((s == 15) & (b < _NB - 1))
        def _():
            pltpu.sync_copy(acc.at[pl.ds(15 * _F_ZR0, _F_ZR15)],
                            out.at[pl.ds(rbase + 15 * _F_ZR0, _F_ZR15)])

        @pl.when((s == 15) & (b == _NB - 1))
        def _():
            last = NUM_UI - (_NB - 1) * _BIN - 15 * _F_ZR0
            pltpu.sync_copy(acc.at[pl.ds(15 * _F_ZR0, last)],
                            out.at[pl.ds(rbase + 15 * _F_ZR0, last)])

        return 0

    lax.fori_loop(0, _NB // 2, do_pass, 0)


def _fh_spmm(msg, cols, vals, rows, bnds, zeros):
    mesh = plsc.VectorSubcoreMesh(core_axis_name="c", subcore_axis_name="s")
    return pl.kernel(
        _fh_spmm_body,
        mesh=mesh,
        out_type=jax.ShapeDtypeStruct((NUM_UI, EMB), jnp.float32),
        scratch_types=[
            pltpu.VMEM_SHARED((_BIN, EMB), jnp.float32),
            pltpu.VMEM((16,), jnp.int32),
            pltpu.VMEM((_F_CH,), jnp.int32),
            pltpu.VMEM((_F_CH,), jnp.float32),
            pltpu.VMEM((_F_CH,), jnp.int32),
            pltpu.VMEM((_F_CH, EMB), jnp.float32),
        ],
    )(msg, cols, vals, rows, bnds, zeros)


def _agg_matmul_body(um_ref, im_ref, w_ref, b_ref, g_ref, msg_ref, gout_ref):
    x = jnp.concatenate([um_ref[...], im_ref[...]], axis=1)
    msg = (
        jax.lax.dot_general(
            x, w_ref[...], (((1,), (0,)), ((), ())),
            preferred_element_type=jnp.float32,
            precision=jax.lax.Precision.HIGHEST,
        )
        + b_ref[...]
    )
    msg_ref[...] = msg
    gout_ref[...] = g_ref[...] + msg


def _agg_matmul(user_msg, item_msg, w, b, g_acc):
    blk = 2000
    grid = (NUM_GROUPS // blk,)
    return pl.pallas_call(
        _agg_matmul_body,
        grid=grid,
        in_specs=[
            pl.BlockSpec((blk, EMB), lambda i: (i, 0)),
            pl.BlockSpec((blk, EMB), lambda i: (i, 0)),
            pl.BlockSpec((2 * EMB, EMB), lambda i: (0, 0)),
            pl.BlockSpec((1, EMB), lambda i: (0, 0)),
            pl.BlockSpec((blk, EMB), lambda i: (i, 0)),
        ],
        out_specs=[
            pl.BlockSpec((blk, EMB), lambda i: (i, 0)),
            pl.BlockSpec((blk, EMB), lambda i: (i, 0)),
        ],
        out_shape=[
            jax.ShapeDtypeStruct((NUM_GROUPS, EMB), jnp.float32),
            jax.ShapeDtypeStruct((NUM_GROUPS, EMB), jnp.float32),
        ],
    )(user_msg, item_msg, w, b, g_acc)


def _ui_sum_body(u_ref, i_ref, e1_ref, e2_ref, o_ref):
    half = pl.program_id(1)
    base = jnp.where(half == 0, u_ref[...], i_ref[...])
    o_ref[...] = base + e1_ref[...] + e2_ref[...]


def _ui_sum(user_emb, item_emb, emb1, emb2):
    blk = 2000
    nb = NUM_USERS // blk
    return pl.pallas_call(
        _ui_sum_body,
        grid=(nb, 2),
        in_specs=[
            pl.BlockSpec((blk, EMB), lambda i, h: (i, 0)),
            pl.BlockSpec((blk, EMB), lambda i, h: (i, 0)),
            pl.BlockSpec((blk, EMB), lambda i, h: (i + h * nb, 0)),
            pl.BlockSpec((blk, EMB), lambda i, h: (i + h * nb, 0)),
        ],
        out_specs=pl.BlockSpec((blk, EMB), lambda i, h: (i + h * nb, 0)),
        out_shape=jax.ShapeDtypeStruct((NUM_UI, EMB), jnp.float32),
    )(user_emb, item_emb, emb1, emb2)


def kernel(user_emb, item_emb, group_emb, W_agg, b_agg,
           uh_row, uh_col, uh_val, ih_row, ih_col, ih_val,
           fh_row, fh_col, fh_val):
    i32 = jnp.int32
    ui0 = jnp.concatenate([user_emb, item_emb], axis=0)
    a_cols = jnp.concatenate([uh_col.astype(i32),
                              ih_col.astype(i32) + NUM_USERS])
    a_vals = jnp.concatenate([uh_val, ih_val])
    a_rows = jnp.concatenate([uh_row.astype(i32), ih_row.astype(i32)])
    perm = jnp.argsort(fh_row.astype(i32))
    f_rows = jnp.concatenate([fh_row.astype(i32)[perm],
                              jnp.full((128,), 2 * NUM_UI, i32)])
    f_cols = jnp.concatenate([fh_col.astype(i32)[perm],
                              jnp.zeros((128,), i32)])
    f_vals = jnp.concatenate([fh_val[perm], jnp.zeros((128,), jnp.float32)])
    bnds = jnp.searchsorted(
        f_rows[:FH_NNZ], jnp.arange(9, dtype=i32) * _BIN).astype(i32)
    bnds = jnp.concatenate([bnds, jnp.zeros((7,), i32)])
    zeros_g = jnp.zeros((640, EMB), jnp.float32)
    zeros_f = jnp.zeros((_F_ZR0, EMB), jnp.float32)

    msgs = _group_spmm(ui0, a_cols, a_vals, a_rows, zeros_g)
    msg, g_acc = _agg_matmul(msgs[0], msgs[1], W_agg[0],
                             b_agg[0].reshape(1, EMB), group_emb)
    emb1 = _fh_spmm(msg, f_cols, f_vals, f_rows, bnds, zeros_f)

    msgs = _group_spmm(emb1, a_cols, a_vals, a_rows, zeros_g)
    msg, g_acc = _agg_matmul(msgs[0], msgs[1], W_agg[1],
                             b_agg[1].reshape(1, EMB), g_acc)
    emb2 = _fh_spmm(msg, f_cols, f_vals, f_rows, bnds, zeros_f)

    final_ui = _ui_sum(user_emb, item_emb, emb1, emb2)
    return (final_ui, g_acc)


# speedup vs baseline: 3.6860x; 1.0853x over previous
import jax
import jax.numpy as jnp
from jax import lax
from jax.experimental import pallas as pl
from jax.experimental.pallas import tpu as pltpu
from jax.experimental.pallas import tpu_sc as plsc

NUM_USERS = 50000
NUM_ITEMS = 50000
NUM_GROUPS = 10000
NUM_UI = NUM_USERS + NUM_ITEMS
EMB = 128
UH_NNZ = 320000
FH_NNZ = 640000

_A_PER_TILE = UH_NNZ // 16
_A_CH = 80
_A_NCH = _A_PER_TILE // _A_CH


_GDN = lax.GatherDimensionNumbers(
    offset_dims=(), collapsed_slice_dims=(0,), start_index_map=(0,))


def _bcast_lane(vec, j):
    idx = jnp.full((16, 1), j, jnp.int32)
    return lax.gather(vec, idx, _GDN, (1,),
                      mode=lax.GatherScatterMode.PROMISE_IN_BOUNDS)


def _scale_chunk(rows_v, val_v, ch, width):
    for g in range(ch // 16):
        vals = val_v[pl.ds(g * 16, 16)]
        for j in range(16):
            vj = _bcast_lane(vals, j)
            jj = g * 16 + j
            for k in range(width // 16):
                sl = pl.ds(k * 16, 16)
                rows_v[jj, sl] = rows_v[jj, sl] * vj


def _group_spmm_body(table, cols, vals, rows, zeros, out,
                     acc, col_v, val_v, row_v, rows_v):
    c = lax.axis_index("c")
    s = lax.axis_index("s")
    @pl.when(s < 15)
    def _():
        pltpu.sync_copy(zeros.at[pl.ds(0, 624)], acc.at[pl.ds(s * 624, 624)])

    @pl.when(s == 15)
    def _():
        pltpu.sync_copy(zeros.at[pl.ds(0, 640)], acc.at[pl.ds(9360, 640)])

    plsc.subcore_barrier()

    base = c * UH_NNZ + s * _A_PER_TILE

    def chunk(i, _):
        lo = base + i * _A_CH
        pltpu.sync_copy(cols.at[pl.ds(lo, _A_CH)], col_v)
        pltpu.sync_copy(vals.at[pl.ds(lo, _A_CH)], val_v)
        pltpu.sync_copy(rows.at[pl.ds(lo, _A_CH)], row_v)
        pltpu.sync_copy(table.at[col_v], rows_v)
        _scale_chunk(rows_v, val_v, _A_CH, EMB)
        pltpu.sync_copy(rows_v, acc.at[row_v], add=True)
        return 0

    lax.fori_loop(0, _A_NCH, chunk, 0)
    plsc.subcore_barrier()

    @pl.when(s < 15)
    def _():
        pltpu.sync_copy(acc.at[pl.ds(s * 624, 624)],
                        out.at[c, pl.ds(s * 624, 624)])

    @pl.when(s == 15)
    def _():
        pltpu.sync_copy(acc.at[pl.ds(9360, 640)],
                        out.at[c, pl.ds(9360, 640)])


def _group_spmm(table, cols, vals, rows, zeros):
    mesh = plsc.VectorSubcoreMesh(core_axis_name="c", subcore_axis_name="s")
    return pl.kernel(
        _group_spmm_body,
        mesh=mesh,
        out_type=jax.ShapeDtypeStruct((2, NUM_GROUPS, EMB), jnp.float32),
        scratch_types=[
            pltpu.VMEM_SHARED((NUM_GROUPS, EMB), jnp.float32),
            pltpu.VMEM((_A_CH,), jnp.int32),
            pltpu.VMEM((_A_CH,), jnp.float32),
            pltpu.VMEM((_A_CH,), jnp.int32),
            pltpu.VMEM((_A_CH, EMB), jnp.float32),
        ],
    )(table, cols, vals, rows, zeros)


_F_CH = 128
_NB = 8
_BIN = 12504
_F_ZR0 = 784
_F_ZR15 = _BIN - 15 * _F_ZR0


def _fh_spmm_body(msg, cols, vals, rows, bnds, zeros, out,
                  acc, bnd_v, col_v, val_v, row_v, rows_v):
    c = lax.axis_index("c")
    s = lax.axis_index("s")
    pltpu.sync_copy(bnds.at[pl.ds(0, 16)], bnd_v)

    def do_pass(p, _):
        b = c * (_NB // 2) + p

        @pl.when(s < 15)
        def _():
            pltpu.sync_copy(zeros.at[pl.ds(0, _F_ZR0)],
                            acc.at[pl.ds(s * _F_ZR0, _F_ZR0)])

        @pl.when(s == 15)
        def _():
            pltpu.sync_copy(zeros.at[pl.ds(0, _F_ZR15)],
                            acc.at[pl.ds(15 * _F_ZR0, _F_ZR15)])

        plsc.subcore_barrier()

        start = bnd_v[pl.ds(b, 1)][0]
        end = bnd_v[pl.ds(b + 1, 1)][0]
        astart = start // 8 * 8
        nch = (end - astart + _F_CH - 1) // _F_CH
        my_nch = (nch - s + 15) // 16
        rbase = b * _BIN

        def chunk(k, _):
            lo = pl.multiple_of(astart + (s + k * 16) * _F_CH, 8)
            pltpu.sync_copy(cols.at[pl.ds(lo, _F_CH)], col_v)
            pltpu.sync_copy(vals.at[pl.ds(lo, _F_CH)], val_v)
            pltpu.sync_copy(rows.at[pl.ds(lo, _F_CH)], row_v)
            for g in range(_F_CH // 16):
                sl = pl.ds(g * 16, 16)
                rel = row_v[sl] - rbase
                inbin = (rel >= 0) & (rel < _BIN)
                val_v[sl] = jnp.where(inbin, val_v[sl], 0.0)
                row_v[sl] = jnp.minimum(jnp.maximum(rel, 0), _BIN - 1)
            pltpu.sync_copy(msg.at[col_v], rows_v)
            _scale_chunk(rows_v, val_v, _F_CH, EMB)
            pltpu.sync_copy(rows_v, acc.at[row_v], add=True)
            return 0

        lax.fori_loop(0, my_nch, chunk, 0)
        plsc.subcore_barrier()

        @pl.when(s < 15)
        def _():
            pltpu.sync_copy(acc.at[pl.ds(s * _F_ZR0, _F_ZR0)],
                            out.at[pl.ds(rbase + s * _F_ZR0, _F_ZR0)])

        @pl.when((s == 15) & (b < _NB - 1))
        def _():
            pltpu.sync_copy(acc.at[pl.ds(15 * _F_ZR0, _F_ZR15)],
                            out.at[pl.ds(rbase + 15 * _F_ZR0, _F_ZR15)])

        @pl.when((s == 15) & (b == _NB - 1))
        def _():
            last = NUM_UI - (_NB - 1) * _BIN - 15 * _F_ZR0
            pltpu.sync_copy(acc.at[pl.ds(15 * _F_ZR0, last)],
                            out.at[pl.ds(rbase + 15 * _F_ZR0, last)])

        return 0

    lax.fori_loop(0, _NB // 2, do_pass, 0)


def _fh_spmm(msg, cols, vals, rows, bnds, zeros):
    mesh = plsc.VectorSubcoreMesh(core_axis_name="c", subcore_axis_name="s")
    return pl.kernel(
        _fh_spmm_body,
        mesh=mesh,
        out_type=jax.ShapeDtypeStruct((NUM_UI, EMB), jnp.float32),
        scratch_types=[
            pltpu.VMEM_SHARED((_BIN, EMB), jnp.float32),
            pltpu.VMEM((16,), jnp.int32),
            pltpu.VMEM((_F_CH,), jnp.int32),
            pltpu.VMEM((_F_CH,), jnp.float32),
            pltpu.VMEM((_F_CH,), jnp.int32),
            pltpu.VMEM((_F_CH, EMB), jnp.float32),
        ],
    )(msg, cols, vals, rows, bnds, zeros)


def _agg_matmul_body(um_ref, im_ref, w_ref, b_ref, g_ref, msg_ref, gout_ref):
    x = jnp.concatenate([um_ref[...], im_ref[...]], axis=1)
    msg = (
        jax.lax.dot_general(
            x, w_ref[...], (((1,), (0,)), ((), ())),
            preferred_element_type=jnp.float32,
            precision=jax.lax.Precision.HIGHEST,
        )
        + b_ref[...]
    )
    msg_ref[...] = msg
    gout_ref[...] = g_ref[...] + msg


def _agg_matmul(user_msg, item_msg, w, b, g_acc):
    blk = 2000
    grid = (NUM_GROUPS // blk,)
    return pl.pallas_call(
        _agg_matmul_body,
        grid=grid,
        in_specs=[
            pl.BlockSpec((blk, EMB), lambda i: (i, 0)),
            pl.BlockSpec((blk, EMB), lambda i: (i, 0)),
            pl.BlockSpec((2 * EMB, EMB), lambda i: (0, 0)),
            pl.BlockSpec((1, EMB), lambda i: (0, 0)),
            pl.BlockSpec((blk, EMB), lambda i: (i, 0)),
        ],
        out_specs=[
            pl.BlockSpec((blk, EMB), lambda i: (i, 0)),
            pl.BlockSpec((blk, EMB), lambda i: (i, 0)),
        ],
        out_shape=[
            jax.ShapeDtypeStruct((NUM_GROUPS, EMB), jnp.float32),
            jax.ShapeDtypeStruct((NUM_GROUPS, EMB), jnp.float32),
        ],
    )(user_msg, item_msg, w, b, g_acc)


def _ui_sum_body(u_ref, i_ref, e1_ref, e2_ref, o_ref):
    half = pl.program_id(1)
    base = jnp.where(half == 0, u_ref[...], i_ref[...])
    o_ref[...] = base + e1_ref[...] + e2_ref[...]


def _ui_sum(user_emb, item_emb, emb1, emb2):
    blk = 2000
    nb = NUM_USERS // blk
    return pl.pallas_call(
        _ui_sum_body,
        grid=(nb, 2),
        in_specs=[
            pl.BlockSpec((blk, EMB), lambda i, h: (i, 0)),
            pl.BlockSpec((blk, EMB), lambda i, h: (i, 0)),
            pl.BlockSpec((blk, EMB), lambda i, h: (i + h * nb, 0)),
            pl.BlockSpec((blk, EMB), lambda i, h: (i + h * nb, 0)),
        ],
        out_specs=pl.BlockSpec((blk, EMB), lambda i, h: (i + h * nb, 0)),
        out_shape=jax.ShapeDtypeStruct((NUM_UI, EMB), jnp.float32),
    )(user_emb, item_emb, emb1, emb2)


def kernel(user_emb, item_emb, group_emb, W_agg, b_agg,
           uh_row, uh_col, uh_val, ih_row, ih_col, ih_val,
           fh_row, fh_col, fh_val):
    i32 = jnp.int32
    ui0 = jnp.concatenate([user_emb, item_emb], axis=0)
    a_cols = jnp.concatenate([uh_col.astype(i32),
                              ih_col.astype(i32) + NUM_USERS])
    a_vals = jnp.concatenate([uh_val, ih_val])
    a_rows = jnp.concatenate([uh_row.astype(i32), ih_row.astype(i32)])
    perm = jnp.argsort(fh_row.astype(i32))
    f_rows = jnp.concatenate([fh_row.astype(i32)[perm],
                              jnp.full((128,), 2 * NUM_UI, i32)])
    f_cols = jnp.concatenate([fh_col.astype(i32)[perm],
                              jnp.zeros((128,), i32)])
    f_vals = jnp.concatenate([fh_val[perm], jnp.zeros((128,), jnp.float32)])
    bnds = jnp.searchsorted(
        f_rows[:FH_NNZ], jnp.arange(9, dtype=i32) * _BIN).astype(i32)
    bnds = jnp.concatenate([bnds, jnp.zeros((7,), i32)])
    zeros_g = jnp.zeros((640, EMB), jnp.float32)
    zeros_f = jnp.zeros((_F_ZR0, EMB), jnp.float32)

    msgs = _group_spmm(ui0, a_cols, a_vals, a_rows, zeros_g)
    msg, g_acc = _agg_matmul(msgs[0], msgs[1], W_agg[0],
                             b_agg[0].reshape(1, EMB), group_emb)
    emb1 = _fh_spmm(msg, f_cols, f_vals, f_rows, bnds, zeros_f)

    msgs = _group_spmm(emb1, a_cols, a_vals, a_rows, zeros_g)
    msg, g_acc = _agg_matmul(msgs[0], msgs[1], W_agg[1],
                             b_agg[1].reshape(1, EMB), g_acc)
    emb2 = _fh_spmm(msg, f_cols, f_vals, f_rows, bnds, zeros_f)

    final_ui = _ui_sum(user_emb, item_emb, emb1, emb2)
    return (final_ui, g_acc)


# speedup vs baseline: 4.0159x; 1.0895x over previous
import jax
import jax.numpy as jnp
from jax import lax
from jax.experimental import pallas as pl
from jax.experimental.pallas import tpu as pltpu
from jax.experimental.pallas import tpu_sc as plsc

NUM_USERS = 50000
NUM_ITEMS = 50000
NUM_GROUPS = 10000
NUM_UI = NUM_USERS + NUM_ITEMS
EMB = 128
UH_NNZ = 320000
FH_NNZ = 640000

_A_CH = 128
_A_NCH = UH_NNZ // _A_CH


_GDN = lax.GatherDimensionNumbers(
    offset_dims=(), collapsed_slice_dims=(0,), start_index_map=(0,))


def _bcast_lane(vec, j):
    idx = jnp.full((16, 1), j, jnp.int32)
    return lax.gather(vec, idx, _GDN, (1,),
                      mode=lax.GatherScatterMode.PROMISE_IN_BOUNDS)


def _scale_chunk(rows_v, val_v, ch, width):
    for g in range(ch // 16):
        vals = val_v[pl.ds(g * 16, 16)]
        for j in range(16):
            vj = _bcast_lane(vals, j)
            jj = g * 16 + j
            for k in range(width // 16):
                sl = pl.ds(k * 16, 16)
                rows_v[jj, sl] = rows_v[jj, sl] * vj


def _group_spmm_body(table, cols, vals, rows, zeros, out,
                     acc, col_v, val_v, row_v, rows_v):
    c = lax.axis_index("c")
    s = lax.axis_index("s")
    @pl.when(s < 15)
    def _():
        pltpu.sync_copy(zeros.at[pl.ds(0, 624)], acc.at[pl.ds(s * 624, 624)])

    @pl.when(s == 15)
    def _():
        pltpu.sync_copy(zeros.at[pl.ds(0, 640)], acc.at[pl.ds(9360, 640)])

    plsc.subcore_barrier()

    base = c * UH_NNZ
    my_nch = (_A_NCH - s + 15) // 16

    def chunk(k, _):
        lo = pl.multiple_of(base + (s + k * 16) * _A_CH, 8)
        pltpu.sync_copy(cols.at[pl.ds(lo, _A_CH)], col_v)
        pltpu.sync_copy(vals.at[pl.ds(lo, _A_CH)], val_v)
        pltpu.sync_copy(rows.at[pl.ds(lo, _A_CH)], row_v)
        pltpu.sync_copy(table.at[col_v], rows_v)
        _scale_chunk(rows_v, val_v, _A_CH, EMB)
        pltpu.sync_copy(rows_v, acc.at[row_v], add=True)
        return 0

    lax.fori_loop(0, my_nch, chunk, 0)
    plsc.subcore_barrier()

    @pl.when(s < 15)
    def _():
        pltpu.sync_copy(acc.at[pl.ds(s * 624, 624)],
                        out.at[c, pl.ds(s * 624, 624)])

    @pl.when(s == 15)
    def _():
        pltpu.sync_copy(acc.at[pl.ds(9360, 640)],
                        out.at[c, pl.ds(9360, 640)])


def _group_spmm(table, cols, vals, rows, zeros):
    mesh = plsc.VectorSubcoreMesh(core_axis_name="c", subcore_axis_name="s")
    return pl.kernel(
        _group_spmm_body,
        mesh=mesh,
        out_type=jax.ShapeDtypeStruct((2, NUM_GROUPS, EMB), jnp.float32),
        scratch_types=[
            pltpu.VMEM_SHARED((NUM_GROUPS, EMB), jnp.float32),
            pltpu.VMEM((_A_CH,), jnp.int32),
            pltpu.VMEM((_A_CH,), jnp.float32),
            pltpu.VMEM((_A_CH,), jnp.int32),
            pltpu.VMEM((_A_CH, EMB), jnp.float32),
        ],
    )(table, cols, vals, rows, zeros)


_F_CH = 128
_NB = 8
_BIN = 12504
_F_ZR0 = 784
_F_ZR15 = _BIN - 15 * _F_ZR0


def _fh_spmm_body(msg, cols, vals, rows, bnds, zeros, out,
                  acc, bnd_v, col_v, val_v, row_v, rows_v):
    c = lax.axis_index("c")
    s = lax.axis_index("s")
    pltpu.sync_copy(bnds.at[pl.ds(0, 16)], bnd_v)

    def do_pass(p, _):
        b = c * (_NB // 2) + p

        @pl.when(s < 15)
        def _():
            pltpu.sync_copy(zeros.at[pl.ds(0, _F_ZR0)],
                            acc.at[pl.ds(s * _F_ZR0, _F_ZR0)])

        @pl.when(s == 15)
        def _():
            pltpu.sync_copy(zeros.at[pl.ds(0, _F_ZR15)],
                            acc.at[pl.ds(15 * _F_ZR0, _F_ZR15)])

        plsc.subcore_barrier()

        start = bnd_v[pl.ds(b, 1)][0]
        end = bnd_v[pl.ds(b + 1, 1)][0]
        astart = start // 8 * 8
        nch = (end - astart + _F_CH - 1) // _F_CH
        my_nch = (nch - s + 15) // 16
        rbase = b * _BIN

        def chunk(k, _):
            lo = pl.multiple_of(astart + (s + k * 16) * _F_CH, 8)
            pltpu.sync_copy(cols.at[pl.ds(lo, _F_CH)], col_v)
            pltpu.sync_copy(vals.at[pl.ds(lo, _F_CH)], val_v)
            pltpu.sync_copy(rows.at[pl.ds(lo, _F_CH)], row_v)
            for g in range(_F_CH // 16):
                sl = pl.ds(g * 16, 16)
                rel = row_v[sl] - rbase
                inbin = (rel >= 0) & (rel < _BIN)
                val_v[sl] = jnp.where(inbin, val_v[sl], 0.0)
                row_v[sl] = jnp.minimum(jnp.maximum(rel, 0), _BIN - 1)
            pltpu.sync_copy(msg.at[col_v], rows_v)
            _scale_chunk(rows_v, val_v, _F_CH, EMB)
            pltpu.sync_copy(rows_v, acc.at[row_v], add=True)
            return 0

        lax.fori_loop(0, my_nch, chunk, 0)
        plsc.subcore_barrier()

        @pl.when(s < 15)
        def _():
            pltpu.sync_copy(acc.at[pl.ds(s * _F_ZR0, _F_ZR0)],
                            out.at[pl.ds(rbase + s * _F_ZR0, _F_ZR0)])

        @pl.when((s == 15) & (b < _NB - 1))
        def _():
            pltpu.sync_copy(acc.at[pl.ds(15 * _F_ZR0, _F_ZR15)],
                            out.at[pl.ds(rbase + 15 * _F_ZR0, _F_ZR15)])

        @pl.when((s == 15) & (b == _NB - 1))
        def _():
            last = NUM_UI - (_NB - 1) * _BIN - 15 * _F_ZR0
            pltpu.sync_copy(acc.at[pl.ds(15 * _F_ZR0, last)],
                            out.at[pl.ds(rbase + 15 * _F_ZR0, last)])

        return 0

    lax.fori_loop(0, _NB // 2, do_pass, 0)


def _fh_spmm(msg, cols, vals, rows, bnds, zeros):
    mesh = plsc.VectorSubcoreMesh(core_axis_name="c", subcore_axis_name="s")
    return pl.kernel(
        _fh_spmm_body,
        mesh=mesh,
        out_type=jax.ShapeDtypeStruct((NUM_UI, EMB), jnp.float32),
        scratch_types=[
            pltpu.VMEM_SHARED((_BIN, EMB), jnp.float32),
            pltpu.VMEM((16,), jnp.int32),
            pltpu.VMEM((_F_CH,), jnp.int32),
            pltpu.VMEM((_F_CH,), jnp.float32),
            pltpu.VMEM((_F_CH,), jnp.int32),
            pltpu.VMEM((_F_CH, EMB), jnp.float32),
        ],
    )(msg, cols, vals, rows, bnds, zeros)


def _agg_matmul_body(um_ref, im_ref, w_ref, b_ref, g_ref, msg_ref, gout_ref):
    x = jnp.concatenate([um_ref[...], im_ref[...]], axis=1)
    msg = (
        jax.lax.dot_general(
            x, w_ref[...], (((1,), (0,)), ((), ())),
            preferred_element_type=jnp.float32,
            precision=jax.lax.Precision.HIGHEST,
        )
        + b_ref[...]
    )
    msg_ref[...] = msg
    gout_ref[...] = g_ref[...] + msg


def _agg_matmul(user_msg, item_msg, w, b, g_acc):
    blk = 2000
    grid = (NUM_GROUPS // blk,)
    return pl.pallas_call(
        _agg_matmul_body,
        grid=grid,
        in_specs=[
            pl.BlockSpec((blk, EMB), lambda i: (i, 0)),
            pl.BlockSpec((blk, EMB), lambda i: (i, 0)),
            pl.BlockSpec((2 * EMB, EMB), lambda i: (0, 0)),
            pl.BlockSpec((1, EMB), lambda i: (0, 0)),
            pl.BlockSpec((blk, EMB), lambda i: (i, 0)),
        ],
        out_specs=[
            pl.BlockSpec((blk, EMB), lambda i: (i, 0)),
            pl.BlockSpec((blk, EMB), lambda i: (i, 0)),
        ],
        out_shape=[
            jax.ShapeDtypeStruct((NUM_GROUPS, EMB), jnp.float32),
            jax.ShapeDtypeStruct((NUM_GROUPS, EMB), jnp.float32),
        ],
    )(user_msg, item_msg, w, b, g_acc)


def _ui_sum_body(u_ref, i_ref, e1_ref, e2_ref, o_ref):
    half = pl.program_id(1)
    base = jnp.where(half == 0, u_ref[...], i_ref[...])
    o_ref[...] = base + e1_ref[...] + e2_ref[...]


def _ui_sum(user_emb, item_emb, emb1, emb2):
    blk = 2000
    nb = NUM_USERS // blk
    return pl.pallas_call(
        _ui_sum_body,
        grid=(nb, 2),
        in_specs=[
            pl.BlockSpec((blk, EMB), lambda i, h: (i, 0)),
            pl.BlockSpec((blk, EMB), lambda i, h: (i, 0)),
            pl.BlockSpec((blk, EMB), lambda i, h: (i + h * nb, 0)),
            pl.BlockSpec((blk, EMB), lambda i, h: (i + h * nb, 0)),
        ],
        out_specs=pl.BlockSpec((blk, EMB), lambda i, h: (i + h * nb, 0)),
        out_shape=jax.ShapeDtypeStruct((NUM_UI, EMB), jnp.float32),
    )(user_emb, item_emb, emb1, emb2)


def kernel(user_emb, item_emb, group_emb, W_agg, b_agg,
           uh_row, uh_col, uh_val, ih_row, ih_col, ih_val,
           fh_row, fh_col, fh_val):
    i32 = jnp.int32
    ui0 = jnp.concatenate([user_emb, item_emb], axis=0)
    a_cols = jnp.concatenate([uh_col.astype(i32),
                              ih_col.astype(i32) + NUM_USERS])
    a_vals = jnp.concatenate([uh_val, ih_val])
    a_rows = jnp.concatenate([uh_row.astype(i32), ih_row.astype(i32)])
    perm = jnp.argsort(fh_row.astype(i32))
    f_rows = jnp.concatenate([fh_row.astype(i32)[perm],
                              jnp.full((128,), 2 * NUM_UI, i32)])
    f_cols = jnp.concatenate([fh_col.astype(i32)[perm],
                              jnp.zeros((128,), i32)])
    f_vals = jnp.concatenate([fh_val[perm], jnp.zeros((128,), jnp.float32)])
    bnds = jnp.searchsorted(
        f_rows[:FH_NNZ], jnp.arange(9, dtype=i32) * _BIN).astype(i32)
    bnds = jnp.concatenate([bnds, jnp.zeros((7,), i32)])
    zeros_g = jnp.zeros((640, EMB), jnp.float32)
    zeros_f = jnp.zeros((_F_ZR0, EMB), jnp.float32)

    msgs = _group_spmm(ui0, a_cols, a_vals, a_rows, zeros_g)
    msg, g_acc = _agg_matmul(msgs[0], msgs[1], W_agg[0],
                             b_agg[0].reshape(1, EMB), group_emb)
    emb1 = _fh_spmm(msg, f_cols, f_vals, f_rows, bnds, zeros_f)

    msgs = _group_spmm(emb1, a_cols, a_vals, a_rows, zeros_g)
    msg, g_acc = _agg_matmul(msgs[0], msgs[1], W_agg[1],
                             b_agg[1].reshape(1, EMB), g_acc)
    emb2 = _fh_spmm(msg, f_cols, f_vals, f_rows, bnds, zeros_f)

    final_ui = _ui_sum(user_emb, item_emb, emb1, emb2)
    return (final_ui, g_acc)
